# baseline (device time: 859360 ns/iter reference)
import jax
import jax.numpy as jnp
from jax import lax
from jax.experimental import pallas as pl
from jax.experimental.pallas import tpu as pltpu

SIZES = [256, 256, 512] + [1024] * 14 + [512, 256, 256]
OFFS = []
_o = 0
for _s in SIZES:
    OFFS.append(_o)
    _o += _s
assert _o == 16384
N = len(SIZES)
MAXROWS = max(SIZES)

S_Y = 6
S_L = 4
S_F = 8
S_YS = 4
S_FS = 4
S_OC = 4
LAG = 2


def kernel(x):
    m, n = x.shape
    half_m = m // 2

    def body(x_ref, out_ref, recv_y, xloc,
             y_send_sems, y_recv_sems, f_send_sems, f_recv_sems,
             load_sems, oc_sems):
        my_x = lax.axis_index("x")
        my_y = lax.axis_index("y")
        my_z = lax.axis_index("z")
        ypeer = (my_x, 1 - my_y, my_z)
        xpeer = (1 - my_x, my_y, my_z)
        a_base = my_x * half_m

        def y_rdma(k):
            return pltpu.make_async_remote_copy(
                src_ref=x_ref.at[pl.ds(a_base + OFFS[k], SIZES[k]), :],
                dst_ref=recv_y.at[k % S_Y, pl.ds(0, SIZES[k]), :],
                send_sem=y_send_sems.at[k % S_YS],
                recv_sem=y_recv_sems.at[k % S_Y],
                device_id=ypeer,
                device_id_type=pl.DeviceIdType.MESH,
            )

        def fwd_rdma(k):
            return pltpu.make_async_remote_copy(
                src_ref=xloc.at[k % S_L, pl.ds(0, SIZES[k]), :],
                dst_ref=out_ref.at[pl.ds(a_base + OFFS[k], SIZES[k]), :],
                send_sem=f_send_sems.at[k % S_FS],
                recv_sem=f_recv_sems.at[k % S_F],
                device_id=xpeer,
                device_id_type=pl.DeviceIdType.MESH,
            )

        def xloc_load(k):
            return pltpu.make_async_copy(
                x_ref.at[pl.ds(a_base + OFFS[k], SIZES[k]), :],
                xloc.at[k % S_L, pl.ds(0, SIZES[k]), :],
                load_sems.at[k % S_L],
            )

        def out_copy(k):
            return pltpu.make_async_copy(
                xloc.at[k % S_L, pl.ds(0, SIZES[k]), :],
                out_ref.at[pl.ds(a_base + OFFS[k], SIZES[k]), :],
                oc_sems.at[k % S_OC],
            )

        bar = pltpu.get_barrier_semaphore()
        for nbr in (ypeer, xpeer):
            pl.semaphore_signal(
                bar, inc=1, device_id=nbr,
                device_id_type=pl.DeviceIdType.MESH,
            )
        pl.semaphore_wait(bar, 2)

        for r in range(N + 6):
            if r < N:
                if r >= S_YS:
                    y_rdma(r - S_YS).wait_send()
                y_rdma(r).start()
                if r >= S_L:
                    fwd_rdma(r - S_L).wait_send()
                    out_copy(r - S_L).wait()
                xloc_load(r).start()
            s = r - LAG
            if 0 <= s < N:
                xloc_load(s).wait()
                y_rdma(s).wait_recv()
                xloc[s % S_L, 0:SIZES[s], :] = (
                    xloc[s % S_L, 0:SIZES[s], :]
                    + recv_y[s % S_Y, 0:SIZES[s], :]
                )
                fwd_rdma(s).start()
                out_copy(s).start()
            t = r - 4
            if 0 <= t < N:
                fwd_rdma(t).wait_recv()

        for k in range(max(0, N - S_YS), N):
            y_rdma(k).wait_send()
        for k in range(max(0, N - S_L), N):
            fwd_rdma(k).wait_send()
            out_copy(k).wait()
        for t in range(max(0, N + 2), N):
            fwd_rdma(t).wait_recv()

    return pl.pallas_call(
        body,
        in_specs=[pl.BlockSpec(memory_space=pltpu.MemorySpace.HBM)],
        out_specs=pl.BlockSpec(memory_space=pltpu.MemorySpace.HBM),
        out_shape=jax.ShapeDtypeStruct((m, n), x.dtype),
        scratch_shapes=[
            pltpu.VMEM((S_Y, MAXROWS, n), x.dtype),
            pltpu.VMEM((S_L, MAXROWS, n), x.dtype),
            pltpu.SemaphoreType.DMA((S_YS,)),
            pltpu.SemaphoreType.DMA((S_Y,)),
            pltpu.SemaphoreType.DMA((S_FS,)),
            pltpu.SemaphoreType.DMA((S_F,)),
            pltpu.SemaphoreType.DMA((S_L,)),
            pltpu.SemaphoreType.DMA((S_OC,)),
        ],
        compiler_params=pltpu.CompilerParams(
            collective_id=0,
            vmem_limit_bytes=60 * 1024 * 1024,
        ),
    )(x)


# device time: 840157 ns/iter; 1.0229x vs baseline; 1.0229x over previous
import jax
import jax.numpy as jnp
from jax import lax
from jax.experimental import pallas as pl
from jax.experimental.pallas import tpu as pltpu

N_PAIRS = 32
N_SLOTS = 6


def kernel(x):
    m, n = x.shape
    pair_m = m // N_PAIRS
    sub_m = pair_m // 2

    def body(x_any_ref, x_prev_ref, out_ref, recv_y, recv_x,
             y_send_sems, y_recv_sems, f_send_sems, f_recv_sems):
        my_x = lax.axis_index("x")
        my_y = lax.axis_index("y")
        my_z = lax.axis_index("z")
        ypeer = (my_x, 1 - my_y, my_z)
        xpeer = (1 - my_x, my_y, my_z)
        p = pl.program_id(0)

        def y_desc(q):
            slot = lax.rem(q, N_SLOTS)
            return pltpu.make_async_remote_copy(
                src_ref=x_any_ref.at[
                    pl.ds(q * pair_m + my_x * sub_m, sub_m), :],
                dst_ref=recv_y.at[slot],
                send_sem=y_send_sems.at[slot],
                recv_sem=y_recv_sems.at[slot],
                device_id=ypeer,
                device_id_type=pl.DeviceIdType.MESH,
            )

        def f_desc(q):
            slot = lax.rem(q, N_SLOTS)
            return pltpu.make_async_remote_copy(
                src_ref=recv_y.at[slot],
                dst_ref=recv_x.at[slot],
                send_sem=f_send_sems.at[slot],
                recv_sem=f_recv_sems.at[slot],
                device_id=xpeer,
                device_id_type=pl.DeviceIdType.MESH,
            )

        @pl.when(p == 0)
        def _():
            bar = pltpu.get_barrier_semaphore()
            for nbr in (ypeer, xpeer):
                pl.semaphore_signal(
                    bar, inc=1, device_id=nbr,
                    device_id_type=pl.DeviceIdType.MESH,
                )
            pl.semaphore_wait(bar, 2)

        @pl.when(p >= 3)
        def _():
            f_desc(p - 3).wait_send()

        @pl.when(p < N_PAIRS)
        def _():
            @pl.when(p >= N_SLOTS)
            def _():
                y_desc(p - N_SLOTS).wait_send()
            y_desc(p).start()

        @pl.when((p >= 1) & (p <= N_PAIRS))
        def _():
            q = p - 1
            y_desc(q).wait_recv()
            f_desc(q).start()

        @pl.when(p >= 3)
        def _():
            r = p - 3
            rslot = lax.rem(r, N_SLOTS)
            off_d = my_x * sub_m
            out_ref[pl.ds(off_d, sub_m), :] = (
                x_prev_ref[pl.ds(off_d, sub_m), :] + recv_y[rslot]
            )
            f_desc(r).wait_recv()
            off_f = (1 - my_x) * sub_m
            out_ref[pl.ds(off_f, sub_m), :] = (
                x_prev_ref[pl.ds(off_f, sub_m), :] + recv_x[rslot]
            )

        @pl.when(p == N_PAIRS + 2)
        def _():
            for k in range(N_SLOTS):
                y_desc(N_PAIRS - N_SLOTS + k).wait_send()

    grid = (N_PAIRS + 3,)
    return pl.pallas_call(
        body,
        grid=grid,
        in_specs=[
            pl.BlockSpec(memory_space=pltpu.MemorySpace.HBM),
            pl.BlockSpec((pair_m, n), lambda i: (jnp.maximum(i - 3, 0), 0)),
        ],
        out_specs=pl.BlockSpec(
            (pair_m, n), lambda i: (jnp.maximum(i - 3, 0), 0)
        ),
        out_shape=jax.ShapeDtypeStruct((m, n), x.dtype),
        scratch_shapes=[
            pltpu.VMEM((N_SLOTS, sub_m, n), x.dtype),
            pltpu.VMEM((N_SLOTS, sub_m, n), x.dtype),
            pltpu.SemaphoreType.DMA((N_SLOTS,)),
            pltpu.SemaphoreType.DMA((N_SLOTS,)),
            pltpu.SemaphoreType.DMA((N_SLOTS,)),
            pltpu.SemaphoreType.DMA((N_SLOTS,)),
        ],
        compiler_params=pltpu.CompilerParams(
            collective_id=0,
            dimension_semantics=("arbitrary",),
            vmem_limit_bytes=60 * 1024 * 1024,
        ),
    )(x, x)


# device time: 682626 ns/iter; 1.2589x vs baseline; 1.2308x over previous
import jax
import jax.numpy as jnp
from jax import lax
from jax.experimental import pallas as pl
from jax.experimental.pallas import tpu as pltpu

NC = 8
S_Y = 6
S_L = 4
S_SND = 4


def kernel(x):
    m, n = x.shape
    q_m = m // 4
    c_m = q_m // NC

    def body(x_ref, out_ref, recv_y, xloc,
             y_send, y_recv, load_sems, oc_sems, sx_send, sz_send,
             rxz_send, rzx_send, xin_sems, zin_sems, xr_sems, zr_sems):
        my_x = lax.axis_index("x")
        my_y = lax.axis_index("y")
        my_z = lax.axis_index("z")
        zbit = lax.rem(my_z, 2)
        ypeer = (my_x, 1 - my_y, my_z)
        xpeer = (1 - my_x, my_y, my_z)
        zpart = (my_x, my_y, my_z + 1 - 2 * zbit)

        o = 2 * my_x + zbit
        ox = lax.rem(o + 2, 4)
        oz = o + 1 - 2 * lax.rem(o, 2)
        od = lax.rem(oz + 2, 4)

        def rows(quarter, k):
            return pl.ds(quarter * q_m + k * c_m, c_m)

        def y_rdma(k):
            return pltpu.make_async_remote_copy(
                src_ref=x_ref.at[rows(o, k), :],
                dst_ref=recv_y.at[k % S_Y],
                send_sem=y_send.at[k % S_SND],
                recv_sem=y_recv.at[k % S_Y],
                device_id=ypeer,
                device_id_type=pl.DeviceIdType.MESH,
            )

        def xload(k):
            return pltpu.make_async_copy(
                x_ref.at[rows(o, k), :],
                xloc.at[k % S_L],
                load_sems.at[k % S_L],
            )

        def out_copy(k):
            return pltpu.make_async_copy(
                xloc.at[k % S_L],
                out_ref.at[rows(o, k), :],
                oc_sems.at[k % S_SND],
            )

        def sum_x(k):
            return pltpu.make_async_remote_copy(
                src_ref=xloc.at[k % S_L],
                dst_ref=out_ref.at[rows(o, k), :],
                send_sem=sx_send.at[k % S_SND],
                recv_sem=xin_sems.at[k],
                device_id=xpeer,
                device_id_type=pl.DeviceIdType.MESH,
            )

        def sum_z(k):
            return pltpu.make_async_remote_copy(
                src_ref=xloc.at[k % S_L],
                dst_ref=out_ref.at[rows(o, k), :],
                send_sem=sz_send.at[k % S_SND],
                recv_sem=zin_sems.at[k],
                device_id=zpart,
                device_id_type=pl.DeviceIdType.MESH,
            )

        def xin(k):
            return pltpu.make_async_remote_copy(
                src_ref=xloc.at[0],
                dst_ref=out_ref.at[rows(ox, k), :],
                send_sem=sx_send.at[0],
                recv_sem=xin_sems.at[k],
                device_id=xpeer,
                device_id_type=pl.DeviceIdType.MESH,
            )

        def zin(k):
            return pltpu.make_async_remote_copy(
                src_ref=xloc.at[0],
                dst_ref=out_ref.at[rows(oz, k), :],
                send_sem=sz_send.at[0],
                recv_sem=zin_sems.at[k],
                device_id=zpart,
                device_id_type=pl.DeviceIdType.MESH,
            )

        def relay_xz(k):
            return pltpu.make_async_remote_copy(
                src_ref=out_ref.at[rows(ox, k), :],
                dst_ref=out_ref.at[rows(ox, k), :],
                send_sem=rxz_send.at[k // 2],
                recv_sem=zr_sems.at[k // 2],
                device_id=zpart,
                device_id_type=pl.DeviceIdType.MESH,
            )

        def relay_zx(k):
            return pltpu.make_async_remote_copy(
                src_ref=out_ref.at[rows(oz, k), :],
                dst_ref=out_ref.at[rows(oz, k), :],
                send_sem=rzx_send.at[k // 2],
                recv_sem=xr_sems.at[k // 2],
                device_id=xpeer,
                device_id_type=pl.DeviceIdType.MESH,
            )

        def xr(k):
            return pltpu.make_async_remote_copy(
                src_ref=out_ref.at[rows(od, k), :],
                dst_ref=out_ref.at[rows(od, k), :],
                send_sem=rzx_send.at[k // 2],
                recv_sem=xr_sems.at[k // 2],
                device_id=xpeer,
                device_id_type=pl.DeviceIdType.MESH,
            )

        def zr(k):
            return pltpu.make_async_remote_copy(
                src_ref=out_ref.at[rows(od, k), :],
                dst_ref=out_ref.at[rows(od, k), :],
                send_sem=rxz_send.at[k // 2],
                recv_sem=zr_sems.at[k // 2],
                device_id=zpart,
                device_id_type=pl.DeviceIdType.MESH,
            )

        bar = pltpu.get_barrier_semaphore()
        for nbr in (ypeer, xpeer, zpart):
            pl.semaphore_signal(
                bar, inc=1, device_id=nbr,
                device_id_type=pl.DeviceIdType.MESH,
            )
        pl.semaphore_wait(bar, 3)

        for r in range(NC + 8):
            if r < NC:
                if r >= S_SND:
                    y_rdma(r - S_SND).wait_send()
                y_rdma(r).start()
                if r >= S_L:
                    sum_x(r - S_L).wait_send()
                    sum_z(r - S_L).wait_send()
                    out_copy(r - S_L).wait()
                xload(r).start()
            s = r - 2
            if 0 <= s < NC:
                xload(s).wait()
                y_rdma(s).wait_recv()
                xloc[s % S_L] = xloc[s % S_L] + recv_y[s % S_Y]
                out_copy(s).start()
                sum_x(s).start()
                sum_z(s).start()
            t = r - 3
            if 0 <= t < NC:
                xin(t).wait_recv()
                if t % 2 == 0:
                    relay_xz(t).start()
                zin(t).wait_recv()
                if t % 2 == 1:
                    relay_zx(t).start()
            u = r - 5
            if 0 <= u < NC:
                if u % 2 == 1:
                    xr(u).wait_recv()
                else:
                    zr(u).wait_recv()

        for k in range(NC - S_SND, NC):
            y_rdma(k).wait_send()
        for k in range(NC - S_L, NC):
            sum_x(k).wait_send()
            sum_z(k).wait_send()
            out_copy(k).wait()
        for k in range(0, NC, 2):
            relay_xz(k).wait_send()
        for k in range(1, NC, 2):
            relay_zx(k).wait_send()

    return pl.pallas_call(
        body,
        in_specs=[pl.BlockSpec(memory_space=pltpu.MemorySpace.HBM)],
        out_specs=pl.BlockSpec(memory_space=pltpu.MemorySpace.HBM),
        out_shape=jax.ShapeDtypeStruct((m, n), x.dtype),
        scratch_shapes=[
            pltpu.VMEM((S_Y, c_m, n), x.dtype),
            pltpu.VMEM((S_L, c_m, n), x.dtype),
            pltpu.SemaphoreType.DMA((S_SND,)),
            pltpu.SemaphoreType.DMA((S_Y,)),
            pltpu.SemaphoreType.DMA((S_L,)),
            pltpu.SemaphoreType.DMA((S_SND,)),
            pltpu.SemaphoreType.DMA((S_SND,)),
            pltpu.SemaphoreType.DMA((S_SND,)),
            pltpu.SemaphoreType.DMA((NC // 2,)),
            pltpu.SemaphoreType.DMA((NC // 2,)),
            pltpu.SemaphoreType.DMA((NC,)),
            pltpu.SemaphoreType.DMA((NC,)),
            pltpu.SemaphoreType.DMA((NC // 2,)),
            pltpu.SemaphoreType.DMA((NC // 2,)),
        ],
        compiler_params=pltpu.CompilerParams(
            collective_id=0,
            vmem_limit_bytes=60 * 1024 * 1024,
        ),
    )(x)


# device time: 660098 ns/iter; 1.3019x vs baseline; 1.0341x over previous
import jax
import jax.numpy as jnp
from jax import lax
from jax.experimental import pallas as pl
from jax.experimental.pallas import tpu as pltpu

NC = 16
S_Y = 6
S_L = 4
S_SND = 4


def kernel(x):
    m, n = x.shape
    q_m = m // 4
    c_m = q_m // NC

    def body(x_ref, out_ref, recv_y, xloc,
             y_send, y_recv, load_sems, oc_sems, sx_send, sz_send,
             rxz_send, rzx_send, xin_sems, zin_sems, xr_sems, zr_sems):
        my_x = lax.axis_index("x")
        my_y = lax.axis_index("y")
        my_z = lax.axis_index("z")
        zbit = lax.rem(my_z, 2)
        ypeer = (my_x, 1 - my_y, my_z)
        xpeer = (1 - my_x, my_y, my_z)
        zpart = (my_x, my_y, my_z + 1 - 2 * zbit)

        o = 2 * my_x + zbit
        ox = lax.rem(o + 2, 4)
        oz = o + 1 - 2 * lax.rem(o, 2)
        od = lax.rem(oz + 2, 4)

        def rows(quarter, k):
            return pl.ds(quarter * q_m + k * c_m, c_m)

        def y_rdma(k):
            return pltpu.make_async_remote_copy(
                src_ref=x_ref.at[rows(o, k), :],
                dst_ref=recv_y.at[k % S_Y],
                send_sem=y_send.at[k % S_SND],
                recv_sem=y_recv.at[k % S_Y],
                device_id=ypeer,
                device_id_type=pl.DeviceIdType.MESH,
            )

        def xload(k):
            return pltpu.make_async_copy(
                x_ref.at[rows(o, k), :],
                xloc.at[k % S_L],
                load_sems.at[k % S_L],
            )

        def out_copy(k):
            return pltpu.make_async_copy(
                xloc.at[k % S_L],
                out_ref.at[rows(o, k), :],
                oc_sems.at[k % S_SND],
            )

        def sum_x(k):
            return pltpu.make_async_remote_copy(
                src_ref=xloc.at[k % S_L],
                dst_ref=out_ref.at[rows(o, k), :],
                send_sem=sx_send.at[k % S_SND],
                recv_sem=xin_sems.at[k],
                device_id=xpeer,
                device_id_type=pl.DeviceIdType.MESH,
            )

        def sum_z(k):
            return pltpu.make_async_remote_copy(
                src_ref=xloc.at[k % S_L],
                dst_ref=out_ref.at[rows(o, k), :],
                send_sem=sz_send.at[k % S_SND],
                recv_sem=zin_sems.at[k],
                device_id=zpart,
                device_id_type=pl.DeviceIdType.MESH,
            )

        def xin(k):
            return pltpu.make_async_remote_copy(
                src_ref=xloc.at[0],
                dst_ref=out_ref.at[rows(ox, k), :],
                send_sem=sx_send.at[0],
                recv_sem=xin_sems.at[k],
                device_id=xpeer,
                device_id_type=pl.DeviceIdType.MESH,
            )

        def zin(k):
            return pltpu.make_async_remote_copy(
                src_ref=xloc.at[0],
                dst_ref=out_ref.at[rows(oz, k), :],
                send_sem=sz_send.at[0],
                recv_sem=zin_sems.at[k],
                device_id=zpart,
                device_id_type=pl.DeviceIdType.MESH,
            )

        def relay_xz(k):
            return pltpu.make_async_remote_copy(
                src_ref=out_ref.at[rows(ox, k), :],
                dst_ref=out_ref.at[rows(ox, k), :],
                send_sem=rxz_send.at[k // 2],
                recv_sem=zr_sems.at[k // 2],
                device_id=zpart,
                device_id_type=pl.DeviceIdType.MESH,
            )

        def relay_zx(k):
            return pltpu.make_async_remote_copy(
                src_ref=out_ref.at[rows(oz, k), :],
                dst_ref=out_ref.at[rows(oz, k), :],
                send_sem=rzx_send.at[k // 2],
                recv_sem=xr_sems.at[k // 2],
                device_id=xpeer,
                device_id_type=pl.DeviceIdType.MESH,
            )

        def xr(k):
            return pltpu.make_async_remote_copy(
                src_ref=out_ref.at[rows(od, k), :],
                dst_ref=out_ref.at[rows(od, k), :],
                send_sem=rzx_send.at[k // 2],
                recv_sem=xr_sems.at[k // 2],
                device_id=xpeer,
                device_id_type=pl.DeviceIdType.MESH,
            )

        def zr(k):
            return pltpu.make_async_remote_copy(
                src_ref=out_ref.at[rows(od, k), :],
                dst_ref=out_ref.at[rows(od, k), :],
                send_sem=rxz_send.at[k // 2],
                recv_sem=zr_sems.at[k // 2],
                device_id=zpart,
                device_id_type=pl.DeviceIdType.MESH,
            )

        bar = pltpu.get_barrier_semaphore()
        for nbr in (ypeer, xpeer, zpart):
            pl.semaphore_signal(
                bar, inc=1, device_id=nbr,
                device_id_type=pl.DeviceIdType.MESH,
            )
        pl.semaphore_wait(bar, 3)

        for r in range(NC + 8):
            if r < NC:
                if r >= S_SND:
                    y_rdma(r - S_SND).wait_send()
                y_rdma(r).start()
                if r >= S_L:
                    sum_x(r - S_L).wait_send()
                    sum_z(r - S_L).wait_send()
                    out_copy(r - S_L).wait()
                xload(r).start()
            s = r - 2
            if 0 <= s < NC:
                xload(s).wait()
                y_rdma(s).wait_recv()
                xloc[s % S_L] = xloc[s % S_L] + recv_y[s % S_Y]
                out_copy(s).start()
                sum_x(s).start()
                sum_z(s).start()
            t = r - 3
            if 0 <= t < NC:
                xin(t).wait_recv()
                if t % 2 == 0:
                    relay_xz(t).start()
                zin(t).wait_recv()
                if t % 2 == 1:
                    relay_zx(t).start()
            u = r - 5
            if 0 <= u < NC:
                if u % 2 == 1:
                    xr(u).wait_recv()
                else:
                    zr(u).wait_recv()

        for k in range(NC - S_SND, NC):
            y_rdma(k).wait_send()
        for k in range(NC - S_L, NC):
            sum_x(k).wait_send()
            sum_z(k).wait_send()
            out_copy(k).wait()
        for k in range(0, NC, 2):
            relay_xz(k).wait_send()
        for k in range(1, NC, 2):
            relay_zx(k).wait_send()

    return pl.pallas_call(
        body,
        in_specs=[pl.BlockSpec(memory_space=pltpu.MemorySpace.HBM)],
        out_specs=pl.BlockSpec(memory_space=pltpu.MemorySpace.HBM),
        out_shape=jax.ShapeDtypeStruct((m, n), x.dtype),
        scratch_shapes=[
            pltpu.VMEM((S_Y, c_m, n), x.dtype),
            pltpu.VMEM((S_L, c_m, n), x.dtype),
            pltpu.SemaphoreType.DMA((S_SND,)),
            pltpu.SemaphoreType.DMA((S_Y,)),
            pltpu.SemaphoreType.DMA((S_L,)),
            pltpu.SemaphoreType.DMA((S_SND,)),
            pltpu.SemaphoreType.DMA((S_SND,)),
            pltpu.SemaphoreType.DMA((S_SND,)),
            pltpu.SemaphoreType.DMA((NC // 2,)),
            pltpu.SemaphoreType.DMA((NC // 2,)),
            pltpu.SemaphoreType.DMA((NC,)),
            pltpu.SemaphoreType.DMA((NC,)),
            pltpu.SemaphoreType.DMA((NC // 2,)),
            pltpu.SemaphoreType.DMA((NC // 2,)),
        ],
        compiler_params=pltpu.CompilerParams(
            collective_id=0,
            vmem_limit_bytes=60 * 1024 * 1024,
        ),
    )(x)
